# Initial kernel scaffold; baseline (speedup 1.0000x reference)
import jax
import jax.numpy as jnp
from jax.experimental import pallas as pl


def kernel(token_ids, weight):
    raise NotImplementedError("write your pallas kernel here")


# speedup vs baseline: 1.8442x; 1.8442x over previous
import functools

import jax
import jax.numpy as jnp
from jax import lax
from jax.experimental import pallas as pl
from jax.experimental.pallas import tpu as pltpu
from jax.experimental.pallas import tpu_sc as plsc

NUM_EMB = 1_000_000
DIM = 64
TOTAL = 16384 * 50
RPS = 128
K = 8
CHUNK = K * RPS
NC, NS = 2, 16
NW = NC * NS
IDX_ROWS = TOTAL // RPS
ROWS_PER_W = IDX_ROWS // NW
NCHUNKS = ROWS_PER_W // K


def _emb_body(idx_hbm, table_hbm, out_hbm, idx_v, rows_v, sem):
    wid = lax.axis_index("s") * NC + lax.axis_index("c")
    row0 = wid * ROWS_PER_W

    def chunk(g, carry):
        base = row0 + g * K
        pltpu.sync_copy(idx_hbm.at[pl.ds(base, K)], idx_v)
        copies = [
            pltpu.async_copy(
                table_hbm.at[idx_v.at[j]],
                rows_v.at[pl.ds(j * RPS, RPS)],
                sem,
            )
            for j in range(K)
        ]
        for c in copies:
            c.wait()
        pltpu.sync_copy(rows_v, out_hbm.at[pl.ds(base * RPS, CHUNK)])
        return carry

    lax.fori_loop(0, NCHUNKS, chunk, 0)


@jax.jit
def kernel(token_ids, weight):
    idx = token_ids.reshape(IDX_ROWS, RPS).astype(jnp.int32)
    mesh = plsc.VectorSubcoreMesh(
        core_axis_name="c", subcore_axis_name="s", num_cores=NC, num_subcores=NS
    )
    out = pl.kernel(
        _emb_body,
        out_type=jax.ShapeDtypeStruct((TOTAL, DIM), jnp.float32),
        mesh=mesh,
        scratch_types=[
            pltpu.VMEM((K, RPS), jnp.int32),
            pltpu.VMEM((CHUNK, DIM), jnp.float32),
            pltpu.SemaphoreType.DMA,
        ],
        compiler_params=pltpu.CompilerParams(use_tc_tiling_on_sc=False),
    )(idx, weight)
    return out.reshape(token_ids.shape + (DIM,))


# speedup vs baseline: 1.8723x; 1.0152x over previous
import jax
import jax.numpy as jnp
from jax import lax
from jax.experimental import pallas as pl
from jax.experimental.pallas import tpu as pltpu
from jax.experimental.pallas import tpu_sc as plsc

NUM_EMB = 1_000_000
DIM = 64
TOTAL = 16384 * 50
RPS = 128
K = 5
CHUNK = K * RPS
NC, NS = 2, 16
NW = NC * NS
IDX_ROWS = TOTAL // RPS
ROWS_PER_W = IDX_ROWS // NW
NCHUNKS = ROWS_PER_W // K


def _emb_body(idx_hbm, table_hbm, out_hbm, idx0, idx1, rows0, rows1,
              si0, si1, sg0, sg1, so0, so1):
    wid = lax.axis_index("s") * NC + lax.axis_index("c")
    row0 = wid * ROWS_PER_W
    idx_v = (idx0, idx1)
    rows_v = (rows0, rows1)
    sem_i = (si0, si1)
    sem_g = (sg0, sg1)
    sem_o = (so0, so1)
    last = row0 + ROWS_PER_W - K

    def start_idx(g, b):
        base = lax.min(row0 + g * K, last)
        pltpu.async_copy(idx_hbm.at[pl.ds(base, K)], idx_v[b], sem_i[b])

    def do_chunk(g, b, wait_out):
        base = row0 + g * K
        pltpu.make_async_copy(idx_hbm.at[pl.ds(row0, K)], idx_v[b], sem_i[b]).wait()
        if wait_out:
            pltpu.make_async_copy(rows_v[b], out_hbm.at[pl.ds(base * RPS, CHUNK)],
                                  sem_o[b]).wait()
        copies = [
            pltpu.async_copy(
                table_hbm.at[idx_v[b].at[j]],
                rows_v[b].at[pl.ds(j * RPS, RPS)],
                sem_g[b],
            )
            for j in range(K)
        ]
        for c in copies:
            c.wait()
        start_idx(g + 2, b)
        pltpu.async_copy(rows_v[b], out_hbm.at[pl.ds(base * RPS, CHUNK)], sem_o[b])

    start_idx(0, 0)
    start_idx(1, 1)
    do_chunk(0, 0, False)
    do_chunk(1, 1, False)

    def pair(i, carry):
        g = 2 * i
        do_chunk(g, 0, True)
        do_chunk(g + 1, 1, True)
        return carry

    lax.fori_loop(1, NCHUNKS // 2, pair, 0)

    for b in range(2):
        pltpu.make_async_copy(rows_v[b], out_hbm.at[pl.ds(row0 * RPS, CHUNK)],
                              sem_o[b]).wait()
        pltpu.make_async_copy(idx_hbm.at[pl.ds(row0, K)], idx_v[b], sem_i[b]).wait()


@jax.jit
def kernel(token_ids, weight):
    idx = token_ids.reshape(IDX_ROWS, RPS).astype(jnp.int32)
    mesh = plsc.VectorSubcoreMesh(
        core_axis_name="c", subcore_axis_name="s", num_cores=NC, num_subcores=NS
    )
    out = pl.kernel(
        _emb_body,
        out_type=jax.ShapeDtypeStruct((TOTAL, DIM), jnp.float32),
        mesh=mesh,
        scratch_types=[
            pltpu.VMEM((K, RPS), jnp.int32),
            pltpu.VMEM((K, RPS), jnp.int32),
            pltpu.VMEM((CHUNK, DIM), jnp.float32),
            pltpu.VMEM((CHUNK, DIM), jnp.float32),
            pltpu.SemaphoreType.DMA,
            pltpu.SemaphoreType.DMA,
            pltpu.SemaphoreType.DMA,
            pltpu.SemaphoreType.DMA,
            pltpu.SemaphoreType.DMA,
            pltpu.SemaphoreType.DMA,
        ],
        compiler_params=pltpu.CompilerParams(use_tc_tiling_on_sc=False),
    )(idx, weight)
    return out.reshape(token_ids.shape + (DIM,))


# speedup vs baseline: 1.8728x; 1.0002x over previous
import jax
import jax.numpy as jnp
from jax import lax
from jax.experimental import pallas as pl
from jax.experimental.pallas import tpu as pltpu
from jax.experimental.pallas import tpu_sc as plsc

NUM_EMB = 1_000_000
DIM = 64
TOTAL = 16384 * 50
RPS = 128
K = 5
CHUNK = K * RPS
NC, NS = 2, 16
NW = NC * NS
IDX_ROWS = TOTAL // RPS
ROWS_PER_W = IDX_ROWS // NW
NCHUNKS = ROWS_PER_W // K


def _emb_body(idx_hbm, table_hbm, out_hbm, idx0, idx1, rows0, rows1,
              si0, si1, sg0, sg1, so0, so1):
    wid = lax.axis_index("s") * NC + lax.axis_index("c")
    row0 = wid * ROWS_PER_W
    idx_v = (idx0, idx1)
    rows_v = (rows0, rows1)
    sem_i = (si0, si1)
    sem_g = (sg0, sg1)
    sem_o = (so0, so1)
    last = row0 + ROWS_PER_W - K

    def start_idx(g, b):
        base = lax.min(row0 + g * K, last)
        pltpu.async_copy(idx_hbm.at[pl.ds(base * RPS, CHUNK)], idx_v[b], sem_i[b])

    def do_chunk(g, b, wait_out):
        base = row0 + g * K
        pltpu.make_async_copy(idx_hbm.at[pl.ds(row0 * RPS, CHUNK)], idx_v[b], sem_i[b]).wait()
        if wait_out:
            pltpu.make_async_copy(rows_v[b], out_hbm.at[pl.ds(base * RPS, CHUNK)],
                                  sem_o[b]).wait()
        pltpu.async_copy(table_hbm.at[idx_v[b]], rows_v[b], sem_g[b]).wait()
        start_idx(g + 2, b)
        pltpu.async_copy(rows_v[b], out_hbm.at[pl.ds(base * RPS, CHUNK)], sem_o[b])

    start_idx(0, 0)
    start_idx(1, 1)
    do_chunk(0, 0, False)
    do_chunk(1, 1, False)

    def pair(i, carry):
        g = 2 * i
        do_chunk(g, 0, True)
        do_chunk(g + 1, 1, True)
        return carry

    lax.fori_loop(1, NCHUNKS // 2, pair, 0)

    for b in range(2):
        pltpu.make_async_copy(rows_v[b], out_hbm.at[pl.ds(row0 * RPS, CHUNK)],
                              sem_o[b]).wait()
        pltpu.make_async_copy(idx_hbm.at[pl.ds(row0 * RPS, CHUNK)], idx_v[b], sem_i[b]).wait()


@jax.jit
def kernel(token_ids, weight):
    idx = token_ids.reshape(TOTAL).astype(jnp.int32)
    mesh = plsc.VectorSubcoreMesh(
        core_axis_name="c", subcore_axis_name="s", num_cores=NC, num_subcores=NS
    )
    out = pl.kernel(
        _emb_body,
        out_type=jax.ShapeDtypeStruct((TOTAL, DIM), jnp.float32),
        mesh=mesh,
        scratch_types=[
            pltpu.VMEM((CHUNK,), jnp.int32),
            pltpu.VMEM((CHUNK,), jnp.int32),
            pltpu.VMEM((CHUNK, DIM), jnp.float32),
            pltpu.VMEM((CHUNK, DIM), jnp.float32),
            pltpu.SemaphoreType.DMA,
            pltpu.SemaphoreType.DMA,
            pltpu.SemaphoreType.DMA,
            pltpu.SemaphoreType.DMA,
            pltpu.SemaphoreType.DMA,
            pltpu.SemaphoreType.DMA,
        ],
        compiler_params=pltpu.CompilerParams(use_tc_tiling_on_sc=False),
    )(idx, weight)
    return out.reshape(token_ids.shape + (DIM,))
